# baseline (speedup 1.0000x reference)
import functools

import jax
import jax.numpy as jnp
from jax import lax
from jax.experimental import pallas as pl
from jax.experimental.pallas import tpu as pltpu
from jax.experimental.pallas import tpu_sc as plsc

SEQ = 200
B = 1024
V = 100000
D = 64
H = 64

NC = 2
NS = 16
NW = NC * NS
CHUNK = 128
PER_STEP = B // CHUNK

NCH = 4
CH_STEPS = SEQ // NCH
CBASE = CH_STEPS // NW
CEXTRA = CH_STEPS - CBASE * NW


BKL = 8192
NBK = (V + BKL - 1) // BKL


def _tc_repack(tbl_t, interpret=False):
    lhs_t = (((0,), (0,)), ((), ()))

    def body(in_ref, eye_ref, out_ref):
        t = jax.lax.dot_general(in_ref[...], eye_ref[...], lhs_t,
                                preferred_element_type=jnp.float32)
        t3 = t.reshape(BKL // 2, 2, D)
        out_ref[...] = jnp.concatenate([t3[:, 0, :], t3[:, 1, :]], axis=1)

    return pl.pallas_call(
        body,
        grid=(NBK,),
        in_specs=[pl.BlockSpec((D, BKL), lambda t: (0, t)),
                  pl.BlockSpec((D, D), lambda t: (0, 0))],
        out_specs=pl.BlockSpec((BKL // 2, 2 * D), lambda t: (t, 0)),
        out_shape=jax.ShapeDtypeStruct((V // 2, 2 * D), jnp.float32),
        interpret=interpret,
    )(tbl_t, jnp.eye(D, dtype=jnp.float32))


def _sc_gather(table, x4, c0):
    mesh = plsc.VectorSubcoreMesh(core_axis_name="c", subcore_axis_name="s")

    @functools.partial(
        pl.kernel,
        out_type=jax.ShapeDtypeStruct((CH_STEPS, B // 2, 2 * D), jnp.float32),
        mesh=mesh,
        scratch_types=[
            pltpu.VMEM((PER_STEP, CHUNK), jnp.int32),
            pltpu.VMEM((B, D), jnp.float32),
            pltpu.SemaphoreType.DMA,
        ],
        compiler_params=pltpu.CompilerParams(use_tc_tiling_on_sc=False),
    )
    def k(table_hbm, idx_hbm, out_hbm, idx_v, rows_v, sem):
        wid = lax.axis_index("s") * NC + lax.axis_index("c")
        t0 = jnp.where(wid < NW - CEXTRA,
                       CBASE * wid,
                       CBASE * wid + (wid - (NW - CEXTRA)))
        t1 = t0 + jnp.where(wid < NW - CEXTRA, CBASE, CBASE + 1)

        @pl.loop(t0, t1)
        def step(t):
            ta = t + c0
            pltpu.sync_copy(idx_hbm.at[ta // 8, :, ta % 8], idx_v)
            copies = [
                pltpu.async_copy(
                    table_hbm.at[idx_v.at[j]],
                    rows_v.at[pl.ds(j * CHUNK, CHUNK)],
                    sem,
                )
                for j in range(PER_STEP)
            ]
            for c in copies:
                c.wait()
            pltpu.sync_copy(rows_v.at[pl.ds(0, B // 2)],
                            out_hbm.at[t, pl.ds(0, B // 2), pl.ds(0, D)])
            pltpu.sync_copy(rows_v.at[pl.ds(B // 2, B // 2)],
                            out_hbm.at[t, pl.ds(0, B // 2), pl.ds(D, D)])

    return k(table, x4)


T_BLK = 10
N_TBLK = CH_STEPS // T_BLK


def _tc_gru(emb128, h_in, w_ih, w_hh, brz, bin_, bhn, interpret=False):
    rhs_t = (((1,), (1,)), ((), ()))

    def body(emb_ref, hin_ref, wih_ref, whh_ref, brz_ref, bin_ref, bhn_ref,
             out_ref, h_ref):
        t = pl.program_id(0)

        @pl.when(t == 0)
        def _():
            h_ref[...] = hin_ref[...]

        wih = wih_ref[...]
        whh = whh_ref[...]
        for i in range(T_BLK):
            h = h_ref[...]
            p = emb_ref[i]
            gi_lo = jax.lax.dot_general(
                wih, p[:, :D], rhs_t, preferred_element_type=jnp.float32)
            gi_hi = jax.lax.dot_general(
                wih, p[:, D:], rhs_t, preferred_element_type=jnp.float32)
            gi = jnp.concatenate([gi_lo, gi_hi], axis=1)
            gh = jnp.dot(whh, h, preferred_element_type=jnp.float32)
            s = gi[: 2 * H] + gh[: 2 * H] + brz_ref[...]
            rz = 0.5 * jnp.tanh(0.5 * s) + 0.5
            r = rz[:H]
            z = rz[H:]
            n = jnp.tanh(gi[2 * H :] + bin_ref[...]
                         + r * (gh[2 * H :] + bhn_ref[...]))
            h_new = n + z * (h - n)
            h_ref[...] = h_new

        @pl.when(t == N_TBLK - 1)
        def _():
            out_ref[...] = h_ref[...]

    return pl.pallas_call(
        body,
        grid=(N_TBLK,),
        in_specs=[
            pl.BlockSpec((T_BLK, B // 2, 2 * D), lambda t: (t, 0, 0)),
            pl.BlockSpec((H, B), lambda t: (0, 0)),
            pl.BlockSpec((3 * H, D), lambda t: (0, 0)),
            pl.BlockSpec((3 * H, H), lambda t: (0, 0)),
            pl.BlockSpec((2 * H, B), lambda t: (0, 0)),
            pl.BlockSpec((H, B), lambda t: (0, 0)),
            pl.BlockSpec((H, B), lambda t: (0, 0)),
        ],
        out_specs=pl.BlockSpec((H, B), lambda t: (0, 0)),
        out_shape=jax.ShapeDtypeStruct((H, B), jnp.float32),
        scratch_shapes=[pltpu.VMEM((H, B), jnp.float32)],
        interpret=interpret,
    )(emb128, h_in, w_ih, w_hh, brz, bin_, bhn)


def kernel(x, table, W_ih, W_hh, b_ih, b_hh):
    x4 = x.astype(jnp.int32).reshape(SEQ // 8, 8, B // CHUNK, CHUNK)
    x4 = x4.transpose(0, 2, 1, 3)
    tbl_t = lax.optimization_barrier(table.T)
    lin_table = _tc_repack(tbl_t).reshape(V, D)
    embs = [_sc_gather(lin_table, x4, c * CH_STEPS) for c in range(NCH)]
    brz = jnp.broadcast_to((b_ih[: 2 * H] + b_hh[: 2 * H])[:, None], (2 * H, B))
    bin_ = jnp.broadcast_to(b_ih[2 * H :][:, None], (H, B))
    bhn = jnp.broadcast_to(b_hh[2 * H :][:, None], (H, B))
    h = jnp.zeros((H, B), jnp.float32)
    for c in range(NCH):
        h = _tc_gru(embs[c], h, W_ih, W_hh, brz, bin_, bhn)
    return h.T[None]


# speedup vs baseline: 1.0271x; 1.0271x over previous
import functools

import jax
import jax.numpy as jnp
from jax import lax
from jax.experimental import pallas as pl
from jax.experimental.pallas import tpu as pltpu
from jax.experimental.pallas import tpu_sc as plsc

SEQ = 200
B = 1024
V = 100000
D = 64
H = 64

NC = 2
NS = 16
NW = NC * NS
CHUNK = 128
PER_STEP = B // CHUNK

NCH = 4
CH_STEPS = SEQ // NCH
CBASE = CH_STEPS // NW
CEXTRA = CH_STEPS - CBASE * NW


BKL = 4096
NBK = (V + BKL - 1) // BKL


def _tc_repack(tbl_t, interpret=False):
    def body(in_ref, out_ref):
        t = in_ref[...].T
        t3 = t.reshape(BKL // 2, 2, D)
        out_ref[...] = jnp.concatenate([t3[:, 0, :], t3[:, 1, :]], axis=1)

    return pl.pallas_call(
        body,
        grid=(NBK,),
        in_specs=[pl.BlockSpec((D, BKL), lambda t: (0, t))],
        out_specs=pl.BlockSpec((BKL // 2, 2 * D), lambda t: (t, 0)),
        out_shape=jax.ShapeDtypeStruct((V // 2, 2 * D), jnp.float32),
        interpret=interpret,
    )(tbl_t)


def _sc_gather(table, x4, c0):
    mesh = plsc.VectorSubcoreMesh(core_axis_name="c", subcore_axis_name="s")

    @functools.partial(
        pl.kernel,
        out_type=jax.ShapeDtypeStruct((CH_STEPS, B // 2, 2 * D), jnp.float32),
        mesh=mesh,
        scratch_types=[
            pltpu.VMEM((PER_STEP, CHUNK), jnp.int32),
            pltpu.VMEM((B, D), jnp.float32),
            pltpu.SemaphoreType.DMA,
        ],
        compiler_params=pltpu.CompilerParams(use_tc_tiling_on_sc=False),
    )
    def k(table_hbm, idx_hbm, out_hbm, idx_v, rows_v, sem):
        wid = lax.axis_index("s") * NC + lax.axis_index("c")
        t0 = jnp.where(wid < NW - CEXTRA,
                       CBASE * wid,
                       CBASE * wid + (wid - (NW - CEXTRA)))
        t1 = t0 + jnp.where(wid < NW - CEXTRA, CBASE, CBASE + 1)

        @pl.loop(t0, t1)
        def step(t):
            ta = t + c0
            pltpu.sync_copy(idx_hbm.at[ta // 8, :, ta % 8], idx_v)
            copies = [
                pltpu.async_copy(
                    table_hbm.at[idx_v.at[j]],
                    rows_v.at[pl.ds(j * CHUNK, CHUNK)],
                    sem,
                )
                for j in range(PER_STEP)
            ]
            for c in copies:
                c.wait()
            pltpu.sync_copy(rows_v.at[pl.ds(0, B // 2)],
                            out_hbm.at[t, pl.ds(0, B // 2), pl.ds(0, D)])
            pltpu.sync_copy(rows_v.at[pl.ds(B // 2, B // 2)],
                            out_hbm.at[t, pl.ds(0, B // 2), pl.ds(D, D)])

    return k(table, x4)


T_BLK = 10
N_TBLK = CH_STEPS // T_BLK


def _tc_gru(emb128, h_in, w_ih, w_hh, brz, bin_, bhn, interpret=False):
    rhs_t = (((1,), (1,)), ((), ()))

    def body(emb_ref, hin_ref, wih_ref, whh_ref, brz_ref, bin_ref, bhn_ref,
             out_ref, h_ref):
        t = pl.program_id(0)

        @pl.when(t == 0)
        def _():
            h_ref[...] = hin_ref[...]

        wih = wih_ref[...]
        whh = whh_ref[...]
        for i in range(T_BLK):
            h = h_ref[...]
            p = emb_ref[i]
            gi_lo = jax.lax.dot_general(
                wih, p[:, :D], rhs_t, preferred_element_type=jnp.float32)
            gi_hi = jax.lax.dot_general(
                wih, p[:, D:], rhs_t, preferred_element_type=jnp.float32)
            gi = jnp.concatenate([gi_lo, gi_hi], axis=1)
            gh = jnp.dot(whh, h, preferred_element_type=jnp.float32)
            s = gi[: 2 * H] + gh[: 2 * H] + brz_ref[...]
            rz = 0.5 * jnp.tanh(0.5 * s) + 0.5
            r = rz[:H]
            z = rz[H:]
            n = jnp.tanh(gi[2 * H :] + bin_ref[...]
                         + r * (gh[2 * H :] + bhn_ref[...]))
            h_new = n + z * (h - n)
            h_ref[...] = h_new

        @pl.when(t == N_TBLK - 1)
        def _():
            out_ref[...] = h_ref[...]

    return pl.pallas_call(
        body,
        grid=(N_TBLK,),
        in_specs=[
            pl.BlockSpec((T_BLK, B // 2, 2 * D), lambda t: (t, 0, 0)),
            pl.BlockSpec((H, B), lambda t: (0, 0)),
            pl.BlockSpec((3 * H, D), lambda t: (0, 0)),
            pl.BlockSpec((3 * H, H), lambda t: (0, 0)),
            pl.BlockSpec((2 * H, B), lambda t: (0, 0)),
            pl.BlockSpec((H, B), lambda t: (0, 0)),
            pl.BlockSpec((H, B), lambda t: (0, 0)),
        ],
        out_specs=pl.BlockSpec((H, B), lambda t: (0, 0)),
        out_shape=jax.ShapeDtypeStruct((H, B), jnp.float32),
        scratch_shapes=[pltpu.VMEM((H, B), jnp.float32)],
        interpret=interpret,
    )(emb128, h_in, w_ih, w_hh, brz, bin_, bhn)


def kernel(x, table, W_ih, W_hh, b_ih, b_hh):
    x4 = x.astype(jnp.int32).reshape(SEQ // 8, 8, B // CHUNK, CHUNK)
    x4 = x4.transpose(0, 2, 1, 3)
    tbl_t = lax.optimization_barrier(table.T)
    lin_table = _tc_repack(tbl_t).reshape(V, D)
    embs = [_sc_gather(lin_table, x4, c * CH_STEPS) for c in range(NCH)]
    brz = jnp.broadcast_to((b_ih[: 2 * H] + b_hh[: 2 * H])[:, None], (2 * H, B))
    bin_ = jnp.broadcast_to(b_ih[2 * H :][:, None], (H, B))
    bhn = jnp.broadcast_to(b_hh[2 * H :][:, None], (H, B))
    h = jnp.zeros((H, B), jnp.float32)
    for c in range(NCH):
        h = _tc_gru(embs[c], h, W_ih, W_hh, brz, bin_, bhn)
    return h.T[None]
